# Initial kernel scaffold; baseline (speedup 1.0000x reference)
import jax
import jax.numpy as jnp
from jax.experimental import pallas as pl


def kernel(x, edge_index, W1, b1, W2, b2):
    raise NotImplementedError("write your pallas kernel here")


# speedup vs baseline: 60.9572x; 60.9572x over previous
import functools

import jax
import jax.numpy as jnp
from jax import lax
from jax.experimental import pallas as pl
from jax.experimental.pallas import tpu as pltpu
from jax.experimental.pallas import tpu_sc as plsc

NC = 2
NS = 16
NW = NC * NS
LANE = 128
KB = 16
EDGE_BLK = KB * LANE
W = 8

_SC_PARAMS = pltpu.CompilerParams(use_tc_tiling_on_sc=False)


def _pad_to(n, m):
  return ((n + m - 1) // m) * m


def _sc_degree(dstm, zeros8, ones8, n_pad, n_iter):
  mesh = plsc.VectorSubcoreMesh(core_axis_name="c", subcore_axis_name="s")
  rows_per = n_pad // NS
  per_tile_rows = n_iter * KB

  @functools.partial(
      pl.kernel,
      mesh=mesh,
      compiler_params=_SC_PARAMS,
      out_type=jax.ShapeDtypeStruct((NC, n_pad, W), jnp.float32),
      scratch_types=[
          pltpu.VMEM((KB, LANE), jnp.int32),
          pltpu.VMEM((LANE, W), jnp.float32),
          pltpu.VMEM_SHARED((n_pad, W), jnp.float32),
      ],
  )
  def k(dstm_hbm, zeros_hbm, ones_hbm, out_hbm, dst_i, ones_v, acc_s):
    c = lax.axis_index("c")
    s = lax.axis_index("s")
    tid = c * NS + s
    base = s * rows_per
    pltpu.sync_copy(zeros_hbm.at[pl.ds(base, rows_per)],
                    acc_s.at[pl.ds(base, rows_per)])
    pltpu.sync_copy(ones_hbm, ones_v)
    plsc.subcore_barrier()

    def body(i, _):
      roff = tid * per_tile_rows + i * KB
      pltpu.sync_copy(dstm_hbm.at[pl.ds(roff, KB)], dst_i)
      for j in range(KB):
        pltpu.sync_copy(ones_v, acc_s.at[dst_i.at[j]], add=True)
      return 0

    lax.fori_loop(0, n_iter, body, 0)
    plsc.subcore_barrier()
    pltpu.sync_copy(acc_s.at[pl.ds(base, rows_per)],
                    out_hbm.at[c, pl.ds(base, rows_per)])

  return k(dstm, zeros8, ones8)


def _sc_aggregate(table, srcm, dstm, zeros8, n_pad, n_iter):
  mesh = plsc.VectorSubcoreMesh(core_axis_name="c", subcore_axis_name="s")
  rows_per = n_pad // NS
  per_tile_rows = n_iter * KB

  @functools.partial(
      pl.kernel,
      mesh=mesh,
      compiler_params=_SC_PARAMS,
      out_type=jax.ShapeDtypeStruct((NC, n_pad, W), jnp.float32),
      scratch_types=[
          pltpu.VMEM((KB, LANE), jnp.int32),
          pltpu.VMEM((KB, LANE), jnp.int32),
          pltpu.VMEM((KB, LANE, W), jnp.float32),
          pltpu.VMEM_SHARED((n_pad, W), jnp.float32),
          pltpu.VMEM_SHARED((n_pad, W), jnp.float32),
          pltpu.SemaphoreType.DMA,
      ],
  )
  def k(table_hbm, srcm_hbm, dstm_hbm, zeros_hbm, out_hbm,
        src_i, dst_i, rows_v, table_s, acc_s, sem):
    c = lax.axis_index("c")
    s = lax.axis_index("s")
    tid = c * NS + s
    base = s * rows_per
    pltpu.sync_copy(table_hbm.at[pl.ds(base, rows_per)],
                    table_s.at[pl.ds(base, rows_per)])
    pltpu.sync_copy(zeros_hbm.at[pl.ds(base, rows_per)],
                    acc_s.at[pl.ds(base, rows_per)])
    plsc.subcore_barrier()

    def body(i, _):
      roff = tid * per_tile_rows + i * KB
      pltpu.sync_copy(srcm_hbm.at[pl.ds(roff, KB)], src_i)
      pltpu.sync_copy(dstm_hbm.at[pl.ds(roff, KB)], dst_i)
      for j in range(KB):
        pltpu.async_copy(table_s.at[src_i.at[j]], rows_v.at[j], sem).wait()
        pltpu.sync_copy(rows_v.at[j], acc_s.at[dst_i.at[j]], add=True)
      return 0

    lax.fori_loop(0, n_iter, body, 0)
    plsc.subcore_barrier()
    pltpu.sync_copy(acc_s.at[pl.ds(base, rows_per)],
                    out_hbm.at[c, pl.ds(base, rows_per)])

  return k(table, srcm, dstm, zeros8)


_BN = 2048


def _mm(t, w_ref):
  kdim = w_ref.shape[0]
  out = t[:, 0:1] * w_ref[0:1, :]
  for kk in range(1, kdim):
    out = out + t[:, kk:kk + 1] * w_ref[kk:kk + 1, :]
  return out


def _tc_stage1(degp, x_pad, n_pad):
  grid = (n_pad // _BN,)

  def body(degp_ref, x_ref, dis_ref, p_ref):
    deg = degp_ref[0, :, 0:1] + degp_ref[1, :, 0:1] + 1.0
    dis = lax.rsqrt(deg)
    dis_ref[...] = dis
    p_ref[...] = jnp.concatenate(
        [dis * x_ref[...], jnp.zeros((x_ref.shape[0], W - 2), jnp.float32)],
        axis=1)

  return pl.pallas_call(
      body,
      grid=grid,
      in_specs=[
          pl.BlockSpec((NC, _BN, W), lambda i: (0, i, 0)),
          pl.BlockSpec((_BN, 2), lambda i: (i, 0)),
      ],
      out_specs=[
          pl.BlockSpec((_BN, 1), lambda i: (i, 0)),
          pl.BlockSpec((_BN, W), lambda i: (i, 0)),
      ],
      out_shape=[
          jax.ShapeDtypeStruct((n_pad, 1), jnp.float32),
          jax.ShapeDtypeStruct((n_pad, W), jnp.float32),
      ],
  )(degp, x_pad)


def _tc_stage2(accp, p8, dis, W1, b1, W2, n_pad):
  grid = (n_pad // _BN,)

  def body(accp_ref, p_ref, dis_ref, w1_ref, b1_ref, w2_ref, q_ref):
    acc = accp_ref[0, :, 0:2] + accp_ref[1, :, 0:2] + p_ref[:, 0:2]
    t = dis_ref[...] * acc
    h = _mm(t, w1_ref) + b1_ref[...]
    h = jnp.maximum(h, 0.0)
    q = dis_ref[...] * _mm(h, w2_ref)
    q_ref[...] = jnp.concatenate(
        [q, jnp.zeros((q.shape[0], W - 2), jnp.float32)], axis=1)

  return pl.pallas_call(
      body,
      grid=grid,
      in_specs=[
          pl.BlockSpec((NC, _BN, W), lambda i: (0, i, 0)),
          pl.BlockSpec((_BN, W), lambda i: (i, 0)),
          pl.BlockSpec((_BN, 1), lambda i: (i, 0)),
          pl.BlockSpec((2, 4), lambda i: (0, 0)),
          pl.BlockSpec((1, 4), lambda i: (0, 0)),
          pl.BlockSpec((4, 2), lambda i: (0, 0)),
      ],
      out_specs=pl.BlockSpec((_BN, W), lambda i: (i, 0)),
      out_shape=jax.ShapeDtypeStruct((n_pad, W), jnp.float32),
  )(accp, p8, dis, W1, b1, W2)


def _tc_stage3(accq, q8, dis, b2, n_pad):
  grid = (n_pad // _BN,)

  def body(accq_ref, q_ref, dis_ref, b2_ref, out_ref):
    acc = accq_ref[0, :, 0:2] + accq_ref[1, :, 0:2] + q_ref[:, 0:2]
    o = dis_ref[...] * acc + b2_ref[...]
    m = jnp.max(o, axis=1, keepdims=True)
    sh = o - m
    lse = jnp.log(jnp.sum(jnp.exp(sh), axis=1, keepdims=True))
    out_ref[...] = sh - lse

  return pl.pallas_call(
      body,
      grid=grid,
      in_specs=[
          pl.BlockSpec((NC, _BN, W), lambda i: (0, i, 0)),
          pl.BlockSpec((_BN, W), lambda i: (i, 0)),
          pl.BlockSpec((_BN, 1), lambda i: (i, 0)),
          pl.BlockSpec((1, 2), lambda i: (0, 0)),
      ],
      out_specs=pl.BlockSpec((_BN, 2), lambda i: (i, 0)),
      out_shape=jax.ShapeDtypeStruct((n_pad, 2), jnp.float32),
  )(accq, q8, dis, b2)


def kernel(x, edge_index, W1, b1, W2, b2):
  n = x.shape[0]
  e = edge_index.shape[1]
  n_pad = _pad_to(n + 1, 2048)
  e_pad = _pad_to(e, NW * EDGE_BLK)
  n_iter = e_pad // (NW * EDGE_BLK)

  src = edge_index[0]
  dst = edge_index[1]
  pad_e = e_pad - e
  src_p = jnp.concatenate([src, jnp.full((pad_e,), n, jnp.int32)])
  dst_p = jnp.concatenate([dst, jnp.full((pad_e,), n, jnp.int32)])
  srcm = src_p.reshape(e_pad // LANE, LANE)
  dstm = dst_p.reshape(e_pad // LANE, LANE)

  x_pad = jnp.zeros((n_pad, 2), jnp.float32).at[:n].set(x)
  zeros8 = jnp.zeros((n_pad, W), jnp.float32)
  ones8 = jnp.zeros((LANE, W), jnp.float32).at[:, 0].set(1.0)

  degp = _sc_degree(dstm, zeros8, ones8, n_pad, n_iter)
  dis, p8 = _tc_stage1(degp, x_pad, n_pad)
  accp = _sc_aggregate(p8, srcm, dstm, zeros8, n_pad, n_iter)
  q8 = _tc_stage2(accp, p8, dis, W1, b1.reshape(1, 4), W2, n_pad)
  accq = _sc_aggregate(q8, srcm, dstm, zeros8, n_pad, n_iter)
  out = _tc_stage3(accq, q8, dis, b2.reshape(1, 2), n_pad)
  return out[:n]


# speedup vs baseline: 79.0115x; 1.2962x over previous
import functools

import jax
import jax.numpy as jnp
from jax import lax
from jax.experimental import pallas as pl
from jax.experimental.pallas import tpu as pltpu
from jax.experimental.pallas import tpu_sc as plsc

NC = 2
NS = 16
NW = NC * NS
LANE = 128
KB = 16
EDGE_BLK = KB * LANE
W = 8

_SC_PARAMS = pltpu.CompilerParams(use_tc_tiling_on_sc=False)


def _pad_to(n, m):
  return ((n + m - 1) // m) * m


def _sc_degree(dstm, zeros8, ones8, n_pad, n_iter):
  mesh = plsc.VectorSubcoreMesh(core_axis_name="c", subcore_axis_name="s")
  rows_per = n_pad // NS
  per_tile_rows = n_iter * KB

  @functools.partial(
      pl.kernel,
      mesh=mesh,
      compiler_params=_SC_PARAMS,
      out_type=jax.ShapeDtypeStruct((NC, n_pad, W), jnp.float32),
      scratch_types=[
          pltpu.VMEM((KB, LANE), jnp.int32),
          pltpu.VMEM((LANE, W), jnp.float32),
          pltpu.VMEM_SHARED((n_pad, W), jnp.float32),
          pltpu.SemaphoreType.DMA,
      ],
  )
  def k(dstm_hbm, zeros_hbm, ones_hbm, out_hbm, dst_i, ones_v, acc_s, sem_s):
    c = lax.axis_index("c")
    s = lax.axis_index("s")
    tid = c * NS + s
    base = s * rows_per
    pltpu.sync_copy(zeros_hbm.at[pl.ds(base, rows_per)],
                    acc_s.at[pl.ds(base, rows_per)])
    pltpu.sync_copy(ones_hbm, ones_v)
    plsc.subcore_barrier()

    def body(i, _):
      roff = tid * per_tile_rows + i * KB
      pltpu.sync_copy(dstm_hbm.at[pl.ds(roff, KB)], dst_i)
      sd = [pltpu.async_copy(ones_v, acc_s.at[dst_i.at[j]], sem_s, add=True)
            for j in range(KB)]
      for d in sd:
        d.wait()
      return 0

    lax.fori_loop(0, n_iter, body, 0)
    plsc.subcore_barrier()
    pltpu.sync_copy(acc_s.at[pl.ds(base, rows_per)],
                    out_hbm.at[c, pl.ds(base, rows_per)])

  return k(dstm, zeros8, ones8)


def _sc_aggregate(table, srcm, dstm, zeros8, n_pad, n_iter):
  mesh = plsc.VectorSubcoreMesh(core_axis_name="c", subcore_axis_name="s")
  rows_per = n_pad // NS
  per_tile_rows = n_iter * KB

  @functools.partial(
      pl.kernel,
      mesh=mesh,
      compiler_params=_SC_PARAMS,
      out_type=jax.ShapeDtypeStruct((NC, n_pad, W), jnp.float32),
      scratch_types=[
          pltpu.VMEM((KB, LANE), jnp.int32),
          pltpu.VMEM((KB, LANE), jnp.int32),
          pltpu.VMEM((KB, LANE, W), jnp.float32),
          pltpu.VMEM_SHARED((n_pad, W), jnp.float32),
          pltpu.VMEM_SHARED((n_pad, W), jnp.float32),
          pltpu.SemaphoreType.DMA,
          pltpu.SemaphoreType.DMA,
          pltpu.SemaphoreType.DMA,
      ],
  )
  def k(table_hbm, srcm_hbm, dstm_hbm, zeros_hbm, out_hbm,
        src_i, dst_i, rows_v, table_s, acc_s, sem_i, sem_g, sem_s):
    c = lax.axis_index("c")
    s = lax.axis_index("s")
    tid = c * NS + s
    base = s * rows_per
    pltpu.sync_copy(table_hbm.at[pl.ds(base, rows_per)],
                    table_s.at[pl.ds(base, rows_per)])
    pltpu.sync_copy(zeros_hbm.at[pl.ds(base, rows_per)],
                    acc_s.at[pl.ds(base, rows_per)])
    plsc.subcore_barrier()

    def body(i, _):
      roff = tid * per_tile_rows + i * KB
      d1 = pltpu.async_copy(srcm_hbm.at[pl.ds(roff, KB)], src_i, sem_i)
      d2 = pltpu.async_copy(dstm_hbm.at[pl.ds(roff, KB)], dst_i, sem_i)
      d1.wait()
      d2.wait()
      gd = [pltpu.async_copy(table_s.at[src_i.at[j]], rows_v.at[j], sem_g)
            for j in range(KB)]
      sd = []
      for j in range(KB):
        gd[j].wait()
        sd.append(pltpu.async_copy(rows_v.at[j], acc_s.at[dst_i.at[j]],
                                   sem_s, add=True))
      for d in sd:
        d.wait()
      return 0

    lax.fori_loop(0, n_iter, body, 0)
    plsc.subcore_barrier()
    pltpu.sync_copy(acc_s.at[pl.ds(base, rows_per)],
                    out_hbm.at[c, pl.ds(base, rows_per)])

  return k(table, srcm, dstm, zeros8)


_BN = 2048


def _mm(t, w_ref):
  kdim = w_ref.shape[0]
  out = t[:, 0:1] * w_ref[0:1, :]
  for kk in range(1, kdim):
    out = out + t[:, kk:kk + 1] * w_ref[kk:kk + 1, :]
  return out


def _tc_stage1(degp, x_pad, n_pad):
  grid = (n_pad // _BN,)

  def body(degp_ref, x_ref, dis_ref, p_ref):
    deg = degp_ref[0, :, 0:1] + degp_ref[1, :, 0:1] + 1.0
    dis = lax.rsqrt(deg)
    dis_ref[...] = dis
    p_ref[...] = jnp.concatenate(
        [dis * x_ref[...], jnp.zeros((x_ref.shape[0], W - 2), jnp.float32)],
        axis=1)

  return pl.pallas_call(
      body,
      grid=grid,
      in_specs=[
          pl.BlockSpec((NC, _BN, W), lambda i: (0, i, 0)),
          pl.BlockSpec((_BN, 2), lambda i: (i, 0)),
      ],
      out_specs=[
          pl.BlockSpec((_BN, 1), lambda i: (i, 0)),
          pl.BlockSpec((_BN, W), lambda i: (i, 0)),
      ],
      out_shape=[
          jax.ShapeDtypeStruct((n_pad, 1), jnp.float32),
          jax.ShapeDtypeStruct((n_pad, W), jnp.float32),
      ],
  )(degp, x_pad)


def _tc_stage2(accp, p8, dis, W1, b1, W2, n_pad):
  grid = (n_pad // _BN,)

  def body(accp_ref, p_ref, dis_ref, w1_ref, b1_ref, w2_ref, q_ref):
    acc = accp_ref[0, :, 0:2] + accp_ref[1, :, 0:2] + p_ref[:, 0:2]
    t = dis_ref[...] * acc
    h = _mm(t, w1_ref) + b1_ref[...]
    h = jnp.maximum(h, 0.0)
    q = dis_ref[...] * _mm(h, w2_ref)
    q_ref[...] = jnp.concatenate(
        [q, jnp.zeros((q.shape[0], W - 2), jnp.float32)], axis=1)

  return pl.pallas_call(
      body,
      grid=grid,
      in_specs=[
          pl.BlockSpec((NC, _BN, W), lambda i: (0, i, 0)),
          pl.BlockSpec((_BN, W), lambda i: (i, 0)),
          pl.BlockSpec((_BN, 1), lambda i: (i, 0)),
          pl.BlockSpec((2, 4), lambda i: (0, 0)),
          pl.BlockSpec((1, 4), lambda i: (0, 0)),
          pl.BlockSpec((4, 2), lambda i: (0, 0)),
      ],
      out_specs=pl.BlockSpec((_BN, W), lambda i: (i, 0)),
      out_shape=jax.ShapeDtypeStruct((n_pad, W), jnp.float32),
  )(accp, p8, dis, W1, b1, W2)


def _tc_stage3(accq, q8, dis, b2, n_pad):
  grid = (n_pad // _BN,)

  def body(accq_ref, q_ref, dis_ref, b2_ref, out_ref):
    acc = accq_ref[0, :, 0:2] + accq_ref[1, :, 0:2] + q_ref[:, 0:2]
    o = dis_ref[...] * acc + b2_ref[...]
    m = jnp.max(o, axis=1, keepdims=True)
    sh = o - m
    lse = jnp.log(jnp.sum(jnp.exp(sh), axis=1, keepdims=True))
    out_ref[...] = sh - lse

  return pl.pallas_call(
      body,
      grid=grid,
      in_specs=[
          pl.BlockSpec((NC, _BN, W), lambda i: (0, i, 0)),
          pl.BlockSpec((_BN, W), lambda i: (i, 0)),
          pl.BlockSpec((_BN, 1), lambda i: (i, 0)),
          pl.BlockSpec((1, 2), lambda i: (0, 0)),
      ],
      out_specs=pl.BlockSpec((_BN, 2), lambda i: (i, 0)),
      out_shape=jax.ShapeDtypeStruct((n_pad, 2), jnp.float32),
  )(accq, q8, dis, b2)


def kernel(x, edge_index, W1, b1, W2, b2):
  n = x.shape[0]
  e = edge_index.shape[1]
  n_pad = _pad_to(n + 1, 2048)
  e_pad = _pad_to(e, NW * EDGE_BLK)
  n_iter = e_pad // (NW * EDGE_BLK)

  src = edge_index[0]
  dst = edge_index[1]
  pad_e = e_pad - e
  src_p = jnp.concatenate([src, jnp.full((pad_e,), n, jnp.int32)])
  dst_p = jnp.concatenate([dst, jnp.full((pad_e,), n, jnp.int32)])
  srcm = src_p.reshape(e_pad // LANE, LANE)
  dstm = dst_p.reshape(e_pad // LANE, LANE)

  x_pad = jnp.zeros((n_pad, 2), jnp.float32).at[:n].set(x)
  zeros8 = jnp.zeros((n_pad, W), jnp.float32)
  ones8 = jnp.zeros((LANE, W), jnp.float32).at[:, 0].set(1.0)

  degp = _sc_degree(dstm, zeros8, ones8, n_pad, n_iter)
  dis, p8 = _tc_stage1(degp, x_pad, n_pad)
  accp = _sc_aggregate(p8, srcm, dstm, zeros8, n_pad, n_iter)
  q8 = _tc_stage2(accp, p8, dis, W1, b1.reshape(1, 4), W2, n_pad)
  accq = _sc_aggregate(q8, srcm, dstm, zeros8, n_pad, n_iter)
  out = _tc_stage3(accq, q8, dis, b2.reshape(1, 2), n_pad)
  return out[:n]
